# Initial kernel scaffold; baseline (speedup 1.0000x reference)
import jax
import jax.numpy as jnp
from jax.experimental import pallas as pl


def kernel(x, edge_index, batch, params, Wc, bc):
    raise NotImplementedError("write your pallas kernel here")


# speedup vs baseline: 3.4528x; 3.4528x over previous
import functools

import jax
import jax.numpy as jnp
from jax import lax
from jax.experimental import pallas as pl
from jax.experimental.pallas import tpu as pltpu
from jax.experimental.pallas import tpu_sc as plsc

NC = 2
NS = 16
CH = 128


def _make_agg(n_nodes, d, e_pad):
  n_workers = NC * NS
  chunks_total = e_pad // CH
  cpw = chunks_total // n_workers
  rps = (n_nodes // NS) // 8 * 8
  rlast = n_nodes - rps * (NS - 1)
  n_acc = n_nodes + 8

  mesh = plsc.VectorSubcoreMesh(
      core_axis_name="c", subcore_axis_name="s", num_cores=NC,
      num_subcores=NS)

  @functools.partial(
      pl.kernel,
      mesh=mesh,
      out_type=jax.ShapeDtypeStruct((NC, n_nodes, d), jnp.float32),
      scratch_types=[
          pltpu.VMEM((CH,), jnp.int32),
          pltpu.VMEM((CH,), jnp.int32),
          pltpu.VMEM((CH, d), jnp.float32),
          pltpu.VMEM_SHARED((n_acc, d), jnp.float32),
          pltpu.SemaphoreType.DMA,
      ],
  )
  def agg(h_hbm, src_hbm, dst_hbm, zeros_hbm, out_hbm,
          src_v, dst_v, rows_v, acc_sh, sem):
    c = lax.axis_index("c")
    s = lax.axis_index("s")
    wid = c * NS + s

    @pl.when(s < NS - 1)
    def _():
      pltpu.sync_copy(zeros_hbm.at[pl.ds(0, rps)], acc_sh.at[pl.ds(s * rps, rps)])

    @pl.when(s == NS - 1)
    def _():
      pltpu.sync_copy(zeros_hbm,
                      acc_sh.at[pl.ds((NS - 1) * rps, rlast + 8)])

    plsc.subcore_barrier()

    def body(k, carry):
      e0 = (wid * cpw + k) * CH
      pltpu.sync_copy(src_hbm.at[pl.ds(e0, CH)], src_v)
      pltpu.sync_copy(dst_hbm.at[pl.ds(e0, CH)], dst_v)
      pltpu.async_copy(h_hbm.at[src_v], rows_v, sem).wait()
      pltpu.sync_copy(rows_v, acc_sh.at[dst_v], add=True)
      return carry

    lax.fori_loop(0, cpw, body, 0)
    plsc.subcore_barrier()

    @pl.when(s < NS - 1)
    def _():
      pltpu.sync_copy(acc_sh.at[pl.ds(s * rps, rps)],
                      out_hbm.at[c].at[pl.ds(s * rps, rps)])

    @pl.when(s == NS - 1)
    def _():
      pltpu.sync_copy(acc_sh.at[pl.ds((NS - 1) * rps, rlast)],
                      out_hbm.at[c].at[pl.ds((NS - 1) * rps, rlast)])

  return agg


def _mlp_call(parts, h, batch3, w1, c1, w2, c2, epsp1, g):
  n, d = h.shape
  br = 1000
  nb = n // br

  def body(eps_ref, p_ref, h_ref, b_ref, w1_ref, c1_ref, w2_ref, c2_ref,
           h_out, pool_out):
    i = pl.program_id(0)
    hb = h_ref[...]
    out = hb * eps_ref[0, 0] + p_ref[0] + p_ref[1]
    z = jnp.dot(out, w1_ref[...], preferred_element_type=jnp.float32,
                precision=lax.Precision.HIGHEST)
    z = jnp.maximum(z + c1_ref[...], 0.0)
    y = jnp.dot(z, w2_ref[...], preferred_element_type=jnp.float32,
                precision=lax.Precision.HIGHEST)
    hn = jnp.maximum(y + c2_ref[...], 0.0)
    h_out[...] = hn
    brow = b_ref[0]
    gids = lax.broadcasted_iota(jnp.int32, (g, br), 0)
    sel = (jnp.broadcast_to(brow, (g, br)) == gids).astype(jnp.float32)
    contrib = jnp.dot(sel, hn, preferred_element_type=jnp.float32,
                      precision=lax.Precision.HIGHEST)

    @pl.when(i == 0)
    def _():
      pool_out[...] = jnp.zeros_like(pool_out)

    pool_out[...] += contrib

  return pl.pallas_call(
      body,
      grid=(nb,),
      in_specs=[
          pl.BlockSpec(memory_space=pltpu.SMEM),
          pl.BlockSpec((NC, br, d), lambda i: (0, i, 0)),
          pl.BlockSpec((br, d), lambda i: (i, 0)),
          pl.BlockSpec((1, 1, br), lambda i: (i, 0, 0)),
          pl.BlockSpec((d, d), lambda i: (0, 0)),
          pl.BlockSpec((1, d), lambda i: (0, 0)),
          pl.BlockSpec((d, d), lambda i: (0, 0)),
          pl.BlockSpec((1, d), lambda i: (0, 0)),
      ],
      out_specs=[
          pl.BlockSpec((br, d), lambda i: (i, 0)),
          pl.BlockSpec((g, d), lambda i: (0, 0)),
      ],
      out_shape=[
          jax.ShapeDtypeStruct((n, d), jnp.float32),
          jax.ShapeDtypeStruct((g, d), jnp.float32),
      ],
  )(epsp1, parts, h, batch3, w1, c1, w2, c2)


def _head_call(pooled, wc3, bcp, g, d, n_layers):
  def body(p_ref, w_ref, b_ref, o_ref):
    acc = jnp.broadcast_to(b_ref[...], (g, d))
    for l in range(n_layers):
      acc = acc + jnp.dot(p_ref[l], w_ref[l],
                          preferred_element_type=jnp.float32,
                          precision=lax.Precision.HIGHEST)
    o_ref[...] = acc

  return pl.pallas_call(
      body,
      out_shape=jax.ShapeDtypeStruct((g, d), jnp.float32),
  )(pooled, wc3, bcp)


def kernel(x, edge_index, batch, params, Wc, bc):
  n, d = x.shape
  e = edge_index.shape[1]
  n_layers = len(params)
  g = 64
  out_dim = Wc.shape[1]

  n_workers = NC * NS
  quantum = n_workers * CH
  e_pad = ((e + quantum - 1) // quantum) * quantum
  pad = e_pad - e
  src = jnp.concatenate([edge_index[0], jnp.zeros((pad,), jnp.int32)])
  dst = jnp.concatenate([edge_index[1], jnp.full((pad,), n, jnp.int32)])
  rps = (n // NS) // 8 * 8
  zeros = jnp.zeros((n - rps * (NS - 1) + 8, d), jnp.float32)
  batch3 = batch.reshape(n // 1000, 1, 1000)

  agg_fn = _make_agg(n, d, e_pad)

  inv = 1.0 / jnp.sqrt(jnp.float32(1.0 + 1e-5))
  h = x
  readouts = []
  for p in params:
    s1 = p["bn1_g"] * inv
    w1 = p["W1"] * s1[None, :]
    c1 = (p["b1"] * s1 + p["bn1_b"]).reshape(1, d)
    s2 = p["bn_g"] * inv
    w2 = p["W2"] * s2[None, :]
    c2 = (p["b2"] * s2 + p["bn_b"]).reshape(1, d)
    epsp1 = (1.0 + p["eps"]).reshape(1, 1).astype(jnp.float32)

    parts = agg_fn(h, src, dst, zeros)
    h, pooled = _mlp_call(parts, h, batch3, w1, c1, w2, c2, epsp1, g)
    readouts.append(pooled)

  pooled_all = jnp.stack(readouts)
  wc3 = jnp.zeros((n_layers, d, d), jnp.float32)
  wc3 = wc3.at[:, :, :out_dim].set(Wc.reshape(n_layers, d, out_dim))
  bcp = jnp.zeros((1, d), jnp.float32).at[0, :out_dim].set(bc)
  logits = _head_call(pooled_all, wc3, bcp, g, d, n_layers)
  return logits[:, :out_dim]


# speedup vs baseline: 4.7192x; 1.3668x over previous
import functools

import jax
import jax.numpy as jnp
from jax import lax
from jax.experimental import pallas as pl
from jax.experimental.pallas import tpu as pltpu
from jax.experimental.pallas import tpu_sc as plsc

NC = 2
NS = 16
CH = 128


def _make_agg(n_nodes, d, e_pad):
  n_workers = NC * NS
  chunks_total = e_pad // CH
  cpw = chunks_total // n_workers
  rps = (n_nodes // NS) // 8 * 8
  rlast = n_nodes - rps * (NS - 1)
  n_acc = n_nodes + 8

  mesh = plsc.VectorSubcoreMesh(
      core_axis_name="c", subcore_axis_name="s", num_cores=NC,
      num_subcores=NS)
  @functools.partial(
      pl.kernel,
      mesh=mesh,
      out_type=jax.ShapeDtypeStruct((NC, n_nodes, d), jnp.float32),
      scratch_types=[
          pltpu.VMEM((CH,), jnp.int32), pltpu.VMEM((CH,), jnp.int32),
          pltpu.VMEM((CH,), jnp.int32), pltpu.VMEM((CH,), jnp.int32),
          pltpu.VMEM((2, CH, d), jnp.float32),
          pltpu.VMEM_SHARED((n_acc, d), jnp.float32),
          pltpu.SemaphoreType.DMA, pltpu.SemaphoreType.DMA,
          pltpu.SemaphoreType.DMA, pltpu.SemaphoreType.DMA,
      ],
  )
  def agg(h_hbm, src_hbm, dst_hbm, zeros_hbm, out_hbm,
          s0, d0, s1, d1, rows_v, acc_sh, gs0, gs1, is0, is1):
    c = lax.axis_index("c")
    s = lax.axis_index("s")
    wid = c * NS + s

    @pl.when(s < NS - 1)
    def _():
      pltpu.sync_copy(zeros_hbm.at[pl.ds(0, rps)], acc_sh.at[pl.ds(s * rps, rps)])

    @pl.when(s == NS - 1)
    def _():
      pltpu.sync_copy(zeros_hbm,
                      acc_sh.at[pl.ds((NS - 1) * rps, rlast + 8)])

    plsc.subcore_barrier()

    sbuf, dbuf = (s0, s1), (d0, d1)
    gsem, isem = (gs0, gs1), (is0, is1)
    base = wid * cpw

    def idx_fetch(k, b):
      e0 = (base + k) * CH
      return (pltpu.async_copy(src_hbm.at[pl.ds(e0, CH)], sbuf[b], isem[b]),
              pltpu.async_copy(dst_hbm.at[pl.ds(e0, CH)], dbuf[b], isem[b]))

    f0 = idx_fetch(0, 0)
    for x in f0:
      x.wait()
    g_pend = pltpu.async_copy(h_hbm.at[sbuf[0]], rows_v.at[0], gs0)
    i_pend = idx_fetch(1, 1)
    for k in range(cpw):
      b = k % 2
      nb = 1 - b
      if k + 1 < cpw:
        for x in i_pend:
          x.wait()
        g_nxt = pltpu.async_copy(h_hbm.at[sbuf[nb]], rows_v.at[nb], gsem[nb])
      else:
        g_nxt = None
      g_pend.wait()
      pltpu.sync_copy(rows_v.at[b], acc_sh.at[dbuf[b]], add=True)
      if k + 2 < cpw:
        i_pend = idx_fetch(k + 2, b)
      g_pend = g_nxt

    plsc.subcore_barrier()

    @pl.when(s < NS - 1)
    def _():
      pltpu.sync_copy(acc_sh.at[pl.ds(s * rps, rps)],
                      out_hbm.at[c].at[pl.ds(s * rps, rps)])

    @pl.when(s == NS - 1)
    def _():
      pltpu.sync_copy(acc_sh.at[pl.ds((NS - 1) * rps, rlast)],
                      out_hbm.at[c].at[pl.ds((NS - 1) * rps, rlast)])

  return agg


def _mlp_call(parts, h, batch3, w1, c1, w2, c2, epsp1, g):
  n, d = h.shape
  br = 1000
  nb = n // br

  def body(eps_ref, p_ref, h_ref, b_ref, w1_ref, c1_ref, w2_ref, c2_ref,
           h_out, pool_out):
    i = pl.program_id(0)
    hb = h_ref[...]
    out = hb * eps_ref[0, 0] + p_ref[0] + p_ref[1]
    z = jnp.dot(out, w1_ref[...], preferred_element_type=jnp.float32,
                precision=lax.Precision.HIGHEST)
    z = jnp.maximum(z + c1_ref[...], 0.0)
    y = jnp.dot(z, w2_ref[...], preferred_element_type=jnp.float32,
                precision=lax.Precision.HIGHEST)
    hn = jnp.maximum(y + c2_ref[...], 0.0)
    h_out[...] = hn
    brow = b_ref[0]
    gids = lax.broadcasted_iota(jnp.int32, (g, br), 0)
    sel = (jnp.broadcast_to(brow, (g, br)) == gids).astype(jnp.float32)
    contrib = jnp.dot(sel, hn, preferred_element_type=jnp.float32,
                      precision=lax.Precision.HIGHEST)

    @pl.when(i == 0)
    def _():
      pool_out[...] = jnp.zeros_like(pool_out)

    pool_out[...] += contrib

  return pl.pallas_call(
      body,
      grid=(nb,),
      in_specs=[
          pl.BlockSpec(memory_space=pltpu.SMEM),
          pl.BlockSpec((NC, br, d), lambda i: (0, i, 0)),
          pl.BlockSpec((br, d), lambda i: (i, 0)),
          pl.BlockSpec((1, 1, br), lambda i: (i, 0, 0)),
          pl.BlockSpec((d, d), lambda i: (0, 0)),
          pl.BlockSpec((1, d), lambda i: (0, 0)),
          pl.BlockSpec((d, d), lambda i: (0, 0)),
          pl.BlockSpec((1, d), lambda i: (0, 0)),
      ],
      out_specs=[
          pl.BlockSpec((br, d), lambda i: (i, 0)),
          pl.BlockSpec((g, d), lambda i: (0, 0)),
      ],
      out_shape=[
          jax.ShapeDtypeStruct((n, d), jnp.float32),
          jax.ShapeDtypeStruct((g, d), jnp.float32),
      ],
  )(epsp1, parts, h, batch3, w1, c1, w2, c2)


def _head_call(pooled, wc3, bcp, g, d, n_layers):
  def body(p_ref, w_ref, b_ref, o_ref):
    acc = jnp.broadcast_to(b_ref[...], (g, d))
    for l in range(n_layers):
      acc = acc + jnp.dot(p_ref[l], w_ref[l],
                          preferred_element_type=jnp.float32,
                          precision=lax.Precision.HIGHEST)
    o_ref[...] = acc

  return pl.pallas_call(
      body,
      out_shape=jax.ShapeDtypeStruct((g, d), jnp.float32),
  )(pooled, wc3, bcp)


def kernel(x, edge_index, batch, params, Wc, bc):
  n, d = x.shape
  e = edge_index.shape[1]
  n_layers = len(params)
  g = 64
  out_dim = Wc.shape[1]

  n_workers = NC * NS
  quantum = n_workers * CH
  e_pad = ((e + quantum - 1) // quantum) * quantum
  pad = e_pad - e
  src = jnp.concatenate([edge_index[0], jnp.zeros((pad,), jnp.int32)])
  dst = jnp.concatenate([edge_index[1], jnp.full((pad,), n, jnp.int32)])
  rps = (n // NS) // 8 * 8
  zeros = jnp.zeros((n - rps * (NS - 1) + 8, d), jnp.float32)
  batch3 = batch.reshape(n // 1000, 1, 1000)

  agg_fn = _make_agg(n, d, e_pad)

  inv = 1.0 / jnp.sqrt(jnp.float32(1.0 + 1e-5))
  h = x
  readouts = []
  for p in params:
    s1 = p["bn1_g"] * inv
    w1 = p["W1"] * s1[None, :]
    c1 = (p["b1"] * s1 + p["bn1_b"]).reshape(1, d)
    s2 = p["bn_g"] * inv
    w2 = p["W2"] * s2[None, :]
    c2 = (p["b2"] * s2 + p["bn_b"]).reshape(1, d)
    epsp1 = (1.0 + p["eps"]).reshape(1, 1).astype(jnp.float32)

    parts = agg_fn(h, src, dst, zeros)
    h, pooled = _mlp_call(parts, h, batch3, w1, c1, w2, c2, epsp1, g)
    readouts.append(pooled)

  pooled_all = jnp.stack(readouts)
  wc3 = jnp.zeros((n_layers, d, d), jnp.float32)
  wc3 = wc3.at[:, :, :out_dim].set(Wc.reshape(n_layers, d, out_dim))
  bcp = jnp.zeros((1, d), jnp.float32).at[0, :out_dim].set(bc)
  logits = _head_call(pooled_all, wc3, bcp, g, d, n_layers)
  return logits[:, :out_dim]


# speedup vs baseline: 4.8362x; 1.0248x over previous
import functools

import jax
import jax.numpy as jnp
from jax import lax
from jax.experimental import pallas as pl
from jax.experimental.pallas import tpu as pltpu
from jax.experimental.pallas import tpu_sc as plsc

NC = 2
NS = 16
CH = 128


def _make_agg(n_nodes, d, e_pad):
  n_workers = NC * NS
  chunks_total = e_pad // CH
  cpw = chunks_total // n_workers
  rps = (n_nodes // NS) // 8 * 8
  rlast = n_nodes - rps * (NS - 1)
  n_acc = n_nodes + 1

  mesh = plsc.VectorSubcoreMesh(
      core_axis_name="c", subcore_axis_name="s", num_cores=NC,
      num_subcores=NS)
  NB = 3
  NI = 6

  @functools.partial(
      pl.kernel,
      mesh=mesh,
      out_type=jax.ShapeDtypeStruct((NC, n_nodes, d), jnp.float32),
      scratch_types=[
          [pltpu.VMEM((CH,), jnp.int32) for _ in range(NI)],
          [pltpu.VMEM((CH,), jnp.int32) for _ in range(NI)],
          pltpu.VMEM((NB, CH, d), jnp.float32),
          pltpu.VMEM_SHARED((n_acc, d), jnp.float32),
          [pltpu.SemaphoreType.DMA for _ in range(NI)],
          [pltpu.SemaphoreType.DMA for _ in range(NB)],
          [pltpu.SemaphoreType.DMA for _ in range(NB)],
      ],
  )
  def agg(h_hbm, src_hbm, dst_hbm, zeros_hbm, out_hbm,
          sbuf, dbuf, rows_v, acc_sh, isem, gsem, ssem):
    c = lax.axis_index("c")
    s = lax.axis_index("s")
    wid = c * NS + s

    @pl.when(s < NS - 1)
    def _():
      pltpu.sync_copy(zeros_hbm.at[pl.ds(0, rps)], acc_sh.at[pl.ds(s * rps, rps)])

    @pl.when(s == NS - 1)
    def _():
      pltpu.sync_copy(zeros_hbm,
                      acc_sh.at[pl.ds((NS - 1) * rps, rlast + 1)])

    plsc.subcore_barrier()

    base = wid * cpw

    def idx_fetch(k):
      j = k % NI
      e0 = (base + k) * CH
      return (pltpu.async_copy(src_hbm.at[pl.ds(e0, CH)], sbuf[j], isem[j]),
              pltpu.async_copy(dst_hbm.at[pl.ds(e0, CH)], dbuf[j], isem[j]))

    idx_pend = {0: idx_fetch(0), 1: idx_fetch(1)}
    for x in idx_pend.pop(0):
      x.wait()
    g_pend = pltpu.async_copy(h_hbm.at[sbuf[0]], rows_v.at[0], gsem[0])
    sc_pend = {}
    for k in range(cpw):
      i = k % NB
      if k + 1 < cpw:
        n1 = (k + 1) % NB
        if k - 2 >= 0:
          sc_pend.pop(k - 2).wait()
        for x in idx_pend.pop(k + 1):
          x.wait()
        g_nxt = pltpu.async_copy(h_hbm.at[sbuf[(k + 1) % NI]],
                                 rows_v.at[n1], gsem[n1])
      else:
        g_nxt = None
      g_pend.wait()
      sc_pend[k] = pltpu.async_copy(rows_v.at[i], acc_sh.at[dbuf[k % NI]],
                                    ssem[i], add=True)
      if k + 2 < cpw:
        idx_pend[k + 2] = idx_fetch(k + 2)
      g_pend = g_nxt
    for k in sorted(sc_pend):
      sc_pend.pop(k).wait()

    plsc.subcore_barrier()

    @pl.when(s < NS - 1)
    def _():
      pltpu.sync_copy(acc_sh.at[pl.ds(s * rps, rps)],
                      out_hbm.at[c].at[pl.ds(s * rps, rps)])

    @pl.when(s == NS - 1)
    def _():
      pltpu.sync_copy(acc_sh.at[pl.ds((NS - 1) * rps, rlast)],
                      out_hbm.at[c].at[pl.ds((NS - 1) * rps, rlast)])

  return agg


def _mlp_call(parts, h, batch3, w1, c1, w2, c2, epsp1, g):
  n, d = h.shape
  br = 1000
  nb = n // br

  def body(eps_ref, p_ref, h_ref, b_ref, w1_ref, c1_ref, w2_ref, c2_ref,
           h_out, pool_out):
    i = pl.program_id(0)
    hb = h_ref[...]
    out = hb * eps_ref[0, 0] + p_ref[0] + p_ref[1]
    z = jnp.dot(out, w1_ref[...], preferred_element_type=jnp.float32,
                precision=lax.Precision.HIGHEST)
    z = jnp.maximum(z + c1_ref[...], 0.0)
    y = jnp.dot(z, w2_ref[...], preferred_element_type=jnp.float32,
                precision=lax.Precision.HIGHEST)
    hn = jnp.maximum(y + c2_ref[...], 0.0)
    h_out[...] = hn
    brow = b_ref[0]
    gids = lax.broadcasted_iota(jnp.int32, (g, br), 0)
    sel = (jnp.broadcast_to(brow, (g, br)) == gids).astype(jnp.float32)
    contrib = jnp.dot(sel, hn, preferred_element_type=jnp.float32,
                      precision=lax.Precision.HIGHEST)

    @pl.when(i == 0)
    def _():
      pool_out[...] = jnp.zeros_like(pool_out)

    pool_out[...] += contrib

  return pl.pallas_call(
      body,
      grid=(nb,),
      in_specs=[
          pl.BlockSpec(memory_space=pltpu.SMEM),
          pl.BlockSpec((NC, br, d), lambda i: (0, i, 0)),
          pl.BlockSpec((br, d), lambda i: (i, 0)),
          pl.BlockSpec((1, 1, br), lambda i: (i, 0, 0)),
          pl.BlockSpec((d, d), lambda i: (0, 0)),
          pl.BlockSpec((1, d), lambda i: (0, 0)),
          pl.BlockSpec((d, d), lambda i: (0, 0)),
          pl.BlockSpec((1, d), lambda i: (0, 0)),
      ],
      out_specs=[
          pl.BlockSpec((br, d), lambda i: (i, 0)),
          pl.BlockSpec((g, d), lambda i: (0, 0)),
      ],
      out_shape=[
          jax.ShapeDtypeStruct((n, d), jnp.float32),
          jax.ShapeDtypeStruct((g, d), jnp.float32),
      ],
  )(epsp1, parts, h, batch3, w1, c1, w2, c2)


def _head_call(pooled, wc3, bcp, g, d, n_layers):
  def body(p_ref, w_ref, b_ref, o_ref):
    acc = jnp.broadcast_to(b_ref[...], (g, d))
    for l in range(n_layers):
      acc = acc + jnp.dot(p_ref[l], w_ref[l],
                          preferred_element_type=jnp.float32,
                          precision=lax.Precision.HIGHEST)
    o_ref[...] = acc

  return pl.pallas_call(
      body,
      out_shape=jax.ShapeDtypeStruct((g, d), jnp.float32),
  )(pooled, wc3, bcp)


def kernel(x, edge_index, batch, params, Wc, bc):
  n, d = x.shape
  e = edge_index.shape[1]
  n_layers = len(params)
  g = 64
  out_dim = Wc.shape[1]

  n_workers = NC * NS
  quantum = n_workers * CH
  e_pad = ((e + quantum - 1) // quantum) * quantum
  pad = e_pad - e
  src = jnp.concatenate([edge_index[0], jnp.zeros((pad,), jnp.int32)])
  dst = jnp.concatenate([edge_index[1], jnp.full((pad,), n, jnp.int32)])
  rps = (n // NS) // 8 * 8
  zeros = jnp.zeros((n - rps * (NS - 1) + 1, d), jnp.float32)
  batch3 = batch.reshape(n // 1000, 1, 1000)

  agg_fn = _make_agg(n, d, e_pad)

  inv = 1.0 / jnp.sqrt(jnp.float32(1.0 + 1e-5))
  h = x
  readouts = []
  for p in params:
    s1 = p["bn1_g"] * inv
    w1 = p["W1"] * s1[None, :]
    c1 = (p["b1"] * s1 + p["bn1_b"]).reshape(1, d)
    s2 = p["bn_g"] * inv
    w2 = p["W2"] * s2[None, :]
    c2 = (p["b2"] * s2 + p["bn_b"]).reshape(1, d)
    epsp1 = (1.0 + p["eps"]).reshape(1, 1).astype(jnp.float32)

    parts = agg_fn(h, src, dst, zeros)
    h, pooled = _mlp_call(parts, h, batch3, w1, c1, w2, c2, epsp1, g)
    readouts.append(pooled)

  pooled_all = jnp.stack(readouts)
  wc3 = jnp.zeros((n_layers, d, d), jnp.float32)
  wc3 = wc3.at[:, :, :out_dim].set(Wc.reshape(n_layers, d, out_dim))
  bcp = jnp.zeros((1, d), jnp.float32).at[0, :out_dim].set(bc)
  logits = _head_call(pooled_all, wc3, bcp, g, d, n_layers)
  return logits[:, :out_dim]


# speedup vs baseline: 4.9530x; 1.0242x over previous
import functools

import jax
import jax.numpy as jnp
from jax import lax
from jax.experimental import pallas as pl
from jax.experimental.pallas import tpu as pltpu
from jax.experimental.pallas import tpu_sc as plsc

NC = 2
NS = 16
CH = 128


def _make_agg(n_nodes, d, e_pad):
  n_workers = NC * NS
  chunks_total = e_pad // CH
  cpw = chunks_total // n_workers
  rps = (n_nodes // NS) // 8 * 8
  rlast = n_nodes - rps * (NS - 1)
  n_acc = n_nodes + 1

  mesh = plsc.VectorSubcoreMesh(
      core_axis_name="c", subcore_axis_name="s", num_cores=NC,
      num_subcores=NS)
  NB = 3
  NI = 6

  @functools.partial(
      pl.kernel,
      mesh=mesh,
      out_type=jax.ShapeDtypeStruct((NC, n_nodes, d), jnp.float32),
      scratch_types=[
          [pltpu.VMEM((CH,), jnp.int32) for _ in range(NI)],
          [pltpu.VMEM((CH,), jnp.int32) for _ in range(NI)],
          pltpu.VMEM((NB, CH, d), jnp.float32),
          pltpu.VMEM_SHARED((n_acc, d), jnp.float32),
          [pltpu.SemaphoreType.DMA for _ in range(NI)],
          [pltpu.SemaphoreType.DMA for _ in range(NB)],
          [pltpu.SemaphoreType.DMA for _ in range(NB)],
      ],
  )
  def agg(h_hbm, src_hbm, dst_hbm, zeros_hbm, out_hbm,
          sbuf, dbuf, rows_v, acc_sh, isem, gsem, ssem):
    c = lax.axis_index("c")
    s = lax.axis_index("s")
    wid = c * NS + s

    @pl.when(s < NS - 1)
    def _():
      pltpu.sync_copy(zeros_hbm.at[pl.ds(0, rps)], acc_sh.at[pl.ds(s * rps, rps)])

    @pl.when(s == NS - 1)
    def _():
      pltpu.sync_copy(zeros_hbm,
                      acc_sh.at[pl.ds((NS - 1) * rps, rlast + 1)])

    plsc.subcore_barrier()

    base = wid * cpw

    def idx_fetch(k):
      j = k % NI
      e0 = (base + k) * CH
      return (pltpu.async_copy(src_hbm.at[pl.ds(e0, CH)], sbuf[j], isem[j]),
              pltpu.async_copy(dst_hbm.at[pl.ds(e0, CH)], dbuf[j], isem[j]))

    def gather(k):
      j = k % NB
      return pltpu.async_copy(h_hbm.at[sbuf[k % NI]], rows_v.at[j], gsem[j])

    idx_pend = {j: idx_fetch(j) for j in range(min(4, cpw))}
    g_pend = {}
    for j in range(min(2, cpw)):
      for x in idx_pend.pop(j):
        x.wait()
      g_pend[j] = gather(j)
    sc_pend = {}
    for k in range(cpw):
      i = k % NB
      if k - 1 >= 0:
        sc_pend.pop(k - 1).wait()
      if k + 2 < cpw:
        for x in idx_pend.pop(k + 2):
          x.wait()
        g_pend[k + 2] = gather(k + 2)
      g_pend.pop(k).wait()
      sc_pend[k] = pltpu.async_copy(rows_v.at[i], acc_sh.at[dbuf[k % NI]],
                                    ssem[i], add=True)
      if k + 4 < cpw:
        idx_pend[k + 4] = idx_fetch(k + 4)
    for k in sorted(sc_pend):
      sc_pend.pop(k).wait()

    plsc.subcore_barrier()

    @pl.when(s < NS - 1)
    def _():
      pltpu.sync_copy(acc_sh.at[pl.ds(s * rps, rps)],
                      out_hbm.at[c].at[pl.ds(s * rps, rps)])

    @pl.when(s == NS - 1)
    def _():
      pltpu.sync_copy(acc_sh.at[pl.ds((NS - 1) * rps, rlast)],
                      out_hbm.at[c].at[pl.ds((NS - 1) * rps, rlast)])

  return agg


def _mlp_call(parts, h, batch3, w1, c1, w2, c2, epsp1, g):
  n, d = h.shape
  br = 1000
  nb = n // br

  def body(eps_ref, p_ref, h_ref, b_ref, w1_ref, c1_ref, w2_ref, c2_ref,
           h_out, pool_out):
    i = pl.program_id(0)
    hb = h_ref[...]
    out = hb * eps_ref[0, 0] + p_ref[0] + p_ref[1]
    z = jnp.dot(out, w1_ref[...], preferred_element_type=jnp.float32,
                precision=lax.Precision.HIGHEST)
    z = jnp.maximum(z + c1_ref[...], 0.0)
    y = jnp.dot(z, w2_ref[...], preferred_element_type=jnp.float32,
                precision=lax.Precision.HIGHEST)
    hn = jnp.maximum(y + c2_ref[...], 0.0)
    h_out[...] = hn
    brow = b_ref[0]
    gids = lax.broadcasted_iota(jnp.int32, (g, br), 0)
    sel = (jnp.broadcast_to(brow, (g, br)) == gids).astype(jnp.float32)
    contrib = jnp.dot(sel, hn, preferred_element_type=jnp.float32,
                      precision=lax.Precision.HIGHEST)

    @pl.when(i == 0)
    def _():
      pool_out[...] = jnp.zeros_like(pool_out)

    pool_out[...] += contrib

  return pl.pallas_call(
      body,
      grid=(nb,),
      in_specs=[
          pl.BlockSpec(memory_space=pltpu.SMEM),
          pl.BlockSpec((NC, br, d), lambda i: (0, i, 0)),
          pl.BlockSpec((br, d), lambda i: (i, 0)),
          pl.BlockSpec((1, 1, br), lambda i: (i, 0, 0)),
          pl.BlockSpec((d, d), lambda i: (0, 0)),
          pl.BlockSpec((1, d), lambda i: (0, 0)),
          pl.BlockSpec((d, d), lambda i: (0, 0)),
          pl.BlockSpec((1, d), lambda i: (0, 0)),
      ],
      out_specs=[
          pl.BlockSpec((br, d), lambda i: (i, 0)),
          pl.BlockSpec((g, d), lambda i: (0, 0)),
      ],
      out_shape=[
          jax.ShapeDtypeStruct((n, d), jnp.float32),
          jax.ShapeDtypeStruct((g, d), jnp.float32),
      ],
  )(epsp1, parts, h, batch3, w1, c1, w2, c2)


def _head_call(pooled, wc3, bcp, g, d, n_layers):
  def body(p_ref, w_ref, b_ref, o_ref):
    acc = jnp.broadcast_to(b_ref[...], (g, d))
    for l in range(n_layers):
      acc = acc + jnp.dot(p_ref[l], w_ref[l],
                          preferred_element_type=jnp.float32,
                          precision=lax.Precision.HIGHEST)
    o_ref[...] = acc

  return pl.pallas_call(
      body,
      out_shape=jax.ShapeDtypeStruct((g, d), jnp.float32),
  )(pooled, wc3, bcp)


def kernel(x, edge_index, batch, params, Wc, bc):
  n, d = x.shape
  e = edge_index.shape[1]
  n_layers = len(params)
  g = 64
  out_dim = Wc.shape[1]

  n_workers = NC * NS
  quantum = n_workers * CH
  e_pad = ((e + quantum - 1) // quantum) * quantum
  pad = e_pad - e
  src = jnp.concatenate([edge_index[0], jnp.zeros((pad,), jnp.int32)])
  dst = jnp.concatenate([edge_index[1], jnp.full((pad,), n, jnp.int32)])
  rps = (n // NS) // 8 * 8
  zeros = jnp.zeros((n - rps * (NS - 1) + 1, d), jnp.float32)
  batch3 = batch.reshape(n // 1000, 1, 1000)

  agg_fn = _make_agg(n, d, e_pad)

  inv = 1.0 / jnp.sqrt(jnp.float32(1.0 + 1e-5))
  h = x
  readouts = []
  for p in params:
    s1 = p["bn1_g"] * inv
    w1 = p["W1"] * s1[None, :]
    c1 = (p["b1"] * s1 + p["bn1_b"]).reshape(1, d)
    s2 = p["bn_g"] * inv
    w2 = p["W2"] * s2[None, :]
    c2 = (p["b2"] * s2 + p["bn_b"]).reshape(1, d)
    epsp1 = (1.0 + p["eps"]).reshape(1, 1).astype(jnp.float32)

    parts = agg_fn(h, src, dst, zeros)
    h, pooled = _mlp_call(parts, h, batch3, w1, c1, w2, c2, epsp1, g)
    readouts.append(pooled)

  pooled_all = jnp.stack(readouts)
  wc3 = jnp.zeros((n_layers, d, d), jnp.float32)
  wc3 = wc3.at[:, :, :out_dim].set(Wc.reshape(n_layers, d, out_dim))
  bcp = jnp.zeros((1, d), jnp.float32).at[0, :out_dim].set(bc)
  logits = _head_call(pooled_all, wc3, bcp, g, d, n_layers)
  return logits[:, :out_dim]
